# baseline (speedup 1.0000x reference)
import functools

import jax
import jax.numpy as jnp
from jax import lax
from jax.experimental import pallas as pl
from jax.experimental.pallas import tpu as pltpu
from jax.experimental.pallas import tpu_sc as plsc


def kernel(x):
    B, C, H, W = x.shape
    R = H * W
    N = B * C
    xf = x.reshape(N, R)

    inds = jnp.linspace(C - 1, 0, C).astype(jnp.int32)
    src = (jnp.arange(B, dtype=jnp.int32)[:, None] * C + inds[None, :]).reshape(N)

    info = plsc.get_sparse_core_info()
    NC, NS = info.num_cores, info.num_subcores
    NW = NC * NS
    rows_per_w = N // NW
    G = 24
    n_chunks = rows_per_w // G

    mesh = plsc.VectorSubcoreMesh(
        core_axis_name="c", subcore_axis_name="s", num_cores=NC
    )

    @functools.partial(
        pl.kernel,
        mesh=mesh,
        out_type=jax.ShapeDtypeStruct((N, R), jnp.float32),
        scratch_types=[
            pltpu.VMEM((rows_per_w,), jnp.int32),
            pltpu.VMEM((G, R), jnp.float32),
            pltpu.SemaphoreType.DMA,
        ],
    )
    def gather_rows(x_hbm, src_hbm, out_hbm, idx_v, rows_v, sem):
        wid = lax.axis_index("s") * NC + lax.axis_index("c")
        base = wid * rows_per_w
        pltpu.sync_copy(src_hbm.at[pl.ds(base, rows_per_w)], idx_v)

        def chunk(j, carry):
            pltpu.async_copy(
                x_hbm.at[idx_v.at[pl.ds(j * G, G)]], rows_v, sem
            ).wait()
            pltpu.sync_copy(rows_v, out_hbm.at[pl.ds(base + j * G, G)])
            return carry

        lax.fori_loop(0, n_chunks, chunk, 0)

    out = gather_rows(xf, src)
    return out.reshape(B, C, H, W)


# speedup vs baseline: 1.0049x; 1.0049x over previous
import functools

import jax
import jax.numpy as jnp
from jax import lax
from jax.experimental import pallas as pl
from jax.experimental.pallas import tpu as pltpu
from jax.experimental.pallas import tpu_sc as plsc


def kernel(x):
    B, C, H, W = x.shape
    R = H * W
    N = B * C
    xf = x.reshape(N, R)

    inds = jnp.linspace(C - 1, 0, C).astype(jnp.int32)
    src = (jnp.arange(B, dtype=jnp.int32)[:, None] * C + inds[None, :]).reshape(N)

    info = plsc.get_sparse_core_info()
    NC, NS = info.num_cores, info.num_subcores
    NW = NC * NS
    rows_per_w = N // NW
    G = 8
    NBUF = 3
    n_chunks = rows_per_w // G
    n_groups = n_chunks // NBUF

    mesh = plsc.VectorSubcoreMesh(
        core_axis_name="c", subcore_axis_name="s", num_cores=NC
    )

    @functools.partial(
        pl.kernel,
        mesh=mesh,
        out_type=jax.ShapeDtypeStruct((N, R), jnp.float32),
        scratch_types=[
            pltpu.VMEM((rows_per_w,), jnp.int32),
            *[pltpu.VMEM((G, R), jnp.float32) for _ in range(NBUF)],
            *[pltpu.SemaphoreType.DMA for _ in range(2 * NBUF)],
        ],
    )
    def gather_rows(x_hbm, src_hbm, out_hbm, idx_v, *scratch):
        bufs = scratch[:NBUF]
        sg = scratch[NBUF : 2 * NBUF]
        ss = scratch[2 * NBUF :]
        wid = lax.axis_index("s") * NC + lax.axis_index("c")
        base = wid * rows_per_w
        pltpu.sync_copy(src_hbm.at[pl.ds(base, rows_per_w)], idx_v)

        def start_gather(m, b):
            pltpu.async_copy(
                x_hbm.at[idx_v.at[pl.ds(m * G, G)]], bufs[b], sg[b]
            )

        def start_store(m, b):
            pltpu.async_copy(
                bufs[b], out_hbm.at[pl.ds(base + m * G, G)], ss[b]
            )

        for b in range(NBUF):
            start_gather(b, b)

        def group(g, carry):
            for b in range(NBUF):
                m = g * NBUF + b
                pltpu.make_async_copy(
                    x_hbm.at[idx_v.at[pl.ds(m * G, G)]], bufs[b], sg[b]
                ).wait()
                start_store(m, b)

                @pl.when(m + NBUF < n_chunks)
                def _():
                    pltpu.make_async_copy(
                        bufs[b], out_hbm.at[pl.ds(base + m * G, G)], ss[b]
                    ).wait()
                    start_gather(m + NBUF, b)

            return carry

        lax.fori_loop(0, n_groups, group, 0)
        for b in range(NBUF):
            pltpu.make_async_copy(
                bufs[b], out_hbm.at[pl.ds(base, G)], ss[b]
            ).wait()

    out = gather_rows(xf, src)
    return out.reshape(B, C, H, W)
